# baseline (speedup 1.0000x reference)
import functools

import jax
import jax.numpy as jnp
from jax import lax
from jax.experimental import pallas as pl
from jax.experimental.pallas import tpu as pltpu
from jax.experimental.pallas import tpu_sc as plsc

T = 1000
TP = 1024
D = 128
B = 16384


def _mlp_body(table_ref, w1_ref, b1_ref, w2_ref, b2_ref, y_ref):
    h = jnp.dot(table_ref[...], w1_ref[...], preferred_element_type=jnp.float32)
    h = h + b1_ref[...]
    h = h * jax.nn.sigmoid(h)
    y = jnp.dot(h, w2_ref[...], preferred_element_type=jnp.float32)
    y_ref[: T, :] = y + b2_ref[...]
    y_ref[T:, :] = jnp.zeros((TP - T, D), jnp.float32)


def _mlp_table(table, W1, b1, W2, b2):
    return pl.pallas_call(
        _mlp_body,
        out_shape=jax.ShapeDtypeStruct((TP, D), jnp.float32),
    )(table, W1, b1.reshape(1, D), W2, b2.reshape(1, D))


def _make_gather():
    info = plsc.get_sparse_core_info()
    nc, ns = info.num_cores, info.num_subcores
    nw = nc * ns
    b_per_w = B // nw
    chunk = 64
    nchunks = b_per_w // chunk
    rows_per_tile = TP // ns
    mesh = plsc.VectorSubcoreMesh(core_axis_name="c", subcore_axis_name="s")

    @functools.partial(
        pl.kernel,
        mesh=mesh,
        out_type=jax.ShapeDtypeStruct((B, D), jnp.float32),
        scratch_types=[
            pltpu.VMEM((b_per_w,), jnp.int32),
            pltpu.VMEM((b_per_w, D), jnp.float32),
            pltpu.VMEM_SHARED((TP, D), jnp.float32),
            pltpu.SemaphoreType.DMA,
        ] + [pltpu.SemaphoreType.DMA] * (b_per_w // 64) + [
            pltpu.SemaphoreType.DMA,
        ],
    )
    def gather_k(y_hbm, idx_hbm, out_hbm, idx_v, rows_v, y_sh,
                 isem, *gw_sems):
        gsems = gw_sems[:-1]
        wsem = gw_sems[-1]
        cid = lax.axis_index("c")
        sid = lax.axis_index("s")
        wid = sid * nc + cid
        base = wid * b_per_w

        def g_src(j):
            return y_hbm if j == 0 else y_sh

        def g_copy(j):
            return pltpu.make_async_copy(
                g_src(j).at[idx_v.at[pl.ds(j * chunk, chunk)]],
                rows_v.at[pl.ds(j * chunk, chunk)],
                gsems[j],
            )

        def w_copy(j):
            return pltpu.make_async_copy(
                rows_v.at[pl.ds(j * chunk, chunk)],
                out_hbm.at[pl.ds(base + j * chunk, chunk)],
                wsem,
            )

        pltpu.async_copy(idx_hbm.at[pl.ds(base, b_per_w)], idx_v, isem)
        r0 = sid * rows_per_tile
        pltpu.sync_copy(
            y_hbm.at[pl.ds(r0, rows_per_tile)], y_sh.at[pl.ds(r0, rows_per_tile)]
        )
        pltpu.make_async_copy(
            idx_hbm.at[pl.ds(base, b_per_w)], idx_v, isem
        ).wait()
        g_copy(0).start()
        plsc.subcore_barrier()
        for j in range(1, nchunks):
            g_copy(j).start()
        for j in range(nchunks):
            g_copy(j).wait()
            w_copy(j).start()
        for j in range(nchunks):
            w_copy(j).wait()

    return gather_k


_gather = _make_gather()


def _epilogue_body(x_ref, o_ref):
    o_ref[...] = x_ref[...]


def _epilogue(out):
    return pl.pallas_call(
        _epilogue_body,
        out_shape=jax.ShapeDtypeStruct((B, D), jnp.float32),
        grid=(1,),
        in_specs=[pl.BlockSpec((8, D), lambda i: (0, 0))],
        out_specs=pl.BlockSpec((8, D), lambda i: (0, 0)),
        input_output_aliases={0: 0},
    )(out)


def kernel(t, table, W1, b1, W2, b2):
    y = _mlp_table(table, W1, b1, W2, b2)
    idx = t.astype(jnp.int32)
    return _epilogue(_gather(y, idx))


# speedup vs baseline: 1.0594x; 1.0594x over previous
import functools

import jax
import jax.numpy as jnp
from jax import lax
from jax.experimental import pallas as pl
from jax.experimental.pallas import tpu as pltpu
from jax.experimental.pallas import tpu_sc as plsc

T = 1000
TP = 1024
D = 128
B = 16384


def _mlp_body(table_ref, w1_ref, b1_ref, w2_ref, b2_ref, y_ref):
    h = jnp.dot(table_ref[...], w1_ref[...], preferred_element_type=jnp.float32)
    h = h + b1_ref[...]
    h = h * jax.nn.sigmoid(h)
    y = jnp.dot(h, w2_ref[...], preferred_element_type=jnp.float32)
    y_ref[: T, :] = y + b2_ref[...]
    y_ref[T:, :] = jnp.zeros((TP - T, D), jnp.float32)


def _mlp_table(table, W1, b1, W2, b2):
    return pl.pallas_call(
        _mlp_body,
        out_shape=jax.ShapeDtypeStruct((TP, D), jnp.float32),
    )(table, W1, b1.reshape(1, D), W2, b2.reshape(1, D))


def _make_gather():
    info = plsc.get_sparse_core_info()
    nc, ns = info.num_cores, info.num_subcores
    nw = nc * ns
    b_per_w = B // nw
    chunk = 64
    nchunks = b_per_w // chunk
    rows_per_tile = TP // ns
    mesh = plsc.VectorSubcoreMesh(core_axis_name="c", subcore_axis_name="s")

    @functools.partial(
        pl.kernel,
        mesh=mesh,
        compiler_params=pltpu.CompilerParams(skip_device_barrier=True),
        out_type=jax.ShapeDtypeStruct((B, D), jnp.float32),
        scratch_types=[
            pltpu.VMEM((b_per_w,), jnp.int32),
            pltpu.VMEM((b_per_w, D), jnp.float32),
            pltpu.VMEM_SHARED((TP, D), jnp.float32),
            pltpu.SemaphoreType.DMA,
        ] + [pltpu.SemaphoreType.DMA] * (b_per_w // 64) + [
            pltpu.SemaphoreType.DMA,
        ],
    )
    def gather_k(y_hbm, idx_hbm, out_hbm, idx_v, rows_v, y_sh,
                 isem, *gw_sems):
        gsems = gw_sems[:-1]
        wsem = gw_sems[-1]
        cid = lax.axis_index("c")
        sid = lax.axis_index("s")
        wid = sid * nc + cid
        base = wid * b_per_w

        def g_src(j):
            return y_hbm if j == 0 else y_sh

        def g_copy(j):
            return pltpu.make_async_copy(
                g_src(j).at[idx_v.at[pl.ds(j * chunk, chunk)]],
                rows_v.at[pl.ds(j * chunk, chunk)],
                gsems[j],
            )

        def w_copy(j):
            return pltpu.make_async_copy(
                rows_v.at[pl.ds(j * chunk, chunk)],
                out_hbm.at[pl.ds(base + j * chunk, chunk)],
                wsem,
            )

        pltpu.async_copy(idx_hbm.at[pl.ds(base, b_per_w)], idx_v, isem)
        r0 = sid * rows_per_tile
        pltpu.sync_copy(
            y_hbm.at[pl.ds(r0, rows_per_tile)], y_sh.at[pl.ds(r0, rows_per_tile)]
        )
        pltpu.make_async_copy(
            idx_hbm.at[pl.ds(base, b_per_w)], idx_v, isem
        ).wait()
        g_copy(0).start()
        plsc.subcore_barrier()
        for j in range(1, nchunks):
            g_copy(j).start()
        for j in range(nchunks):
            g_copy(j).wait()
            w_copy(j).start()
        for j in range(nchunks):
            w_copy(j).wait()

    return gather_k


_gather = _make_gather()


def kernel(t, table, W1, b1, W2, b2):
    y = _mlp_table(table, W1, b1, W2, b2)
    idx = t.astype(jnp.int32)
    return _gather(y, idx)


# speedup vs baseline: 1.0612x; 1.0016x over previous
import functools

import jax
import jax.numpy as jnp
from jax import lax
from jax.experimental import pallas as pl
from jax.experimental.pallas import tpu as pltpu
from jax.experimental.pallas import tpu_sc as plsc

T = 1000
TP = 1024
D = 128
B = 16384


def _mlp_body(table_ref, w1_ref, b1_ref, w2_ref, b2_ref, y_ref):
    h = jnp.dot(table_ref[...], w1_ref[...], preferred_element_type=jnp.float32)
    h = h + b1_ref[...]
    h = h * jax.nn.sigmoid(h)
    y = jnp.dot(h, w2_ref[...], preferred_element_type=jnp.float32)
    y_ref[: T, :] = y + b2_ref[...]
    y_ref[T:, :] = jnp.zeros((TP - T, D), jnp.float32)


def _mlp_table(table, W1, b1, W2, b2):
    return pl.pallas_call(
        _mlp_body,
        out_shape=jax.ShapeDtypeStruct((TP, D), jnp.float32),
    )(table, W1, b1.reshape(1, D), W2, b2.reshape(1, D))


def _make_gather():
    info = plsc.get_sparse_core_info()
    nc, ns = info.num_cores, info.num_subcores
    nw = nc * ns
    b_per_w = B // nw
    chunk = 64
    nchunks = b_per_w // chunk
    rows_per_tile = TP // ns
    mesh = plsc.VectorSubcoreMesh(core_axis_name="c", subcore_axis_name="s")

    @functools.partial(
        pl.kernel,
        mesh=mesh,
        compiler_params=pltpu.CompilerParams(skip_device_barrier=True),
        out_type=jax.ShapeDtypeStruct((B, D), jnp.float32),
        scratch_types=[
            pltpu.VMEM((b_per_w,), jnp.int32),
            pltpu.VMEM((b_per_w, D), jnp.float32),
            pltpu.VMEM_SHARED((TP, D), jnp.float32),
            pltpu.SemaphoreType.DMA,
        ] + [pltpu.SemaphoreType.DMA] * (b_per_w // 64) + [
            pltpu.SemaphoreType.DMA,
        ],
    )
    def gather_k(y_hbm, idx_hbm, out_hbm, idx_v, rows_v, y_sh,
                 isem, *gw_sems):
        gsems = gw_sems[:-1]
        wsem = gw_sems[-1]
        cid = lax.axis_index("c")
        sid = lax.axis_index("s")
        wid = sid * nc + cid
        base = wid * b_per_w

        def g_src(j):
            return y_sh

        def g_copy(j):
            return pltpu.make_async_copy(
                g_src(j).at[idx_v.at[pl.ds(j * chunk, chunk)]],
                rows_v.at[pl.ds(j * chunk, chunk)],
                gsems[j],
            )

        def w_copy(j):
            return pltpu.make_async_copy(
                rows_v.at[pl.ds(j * chunk, chunk)],
                out_hbm.at[pl.ds(base + j * chunk, chunk)],
                wsem,
            )

        pltpu.async_copy(idx_hbm.at[pl.ds(base, b_per_w)], idx_v, isem)
        r0 = sid * rows_per_tile
        pltpu.sync_copy(
            y_hbm.at[pl.ds(r0, rows_per_tile)], y_sh.at[pl.ds(r0, rows_per_tile)]
        )
        pltpu.make_async_copy(
            idx_hbm.at[pl.ds(base, b_per_w)], idx_v, isem
        ).wait()
        plsc.subcore_barrier()
        for j in range(nchunks):
            g_copy(j).start()
        for j in range(nchunks):
            g_copy(j).wait()
            w_copy(j).start()
        for j in range(nchunks):
            w_copy(j).wait()

    return gather_k


_gather = _make_gather()


def kernel(t, table, W1, b1, W2, b2):
    y = _mlp_table(table, W1, b1, W2, b2)
    idx = t.astype(jnp.int32)
    return _gather(y, idx)
